# Initial kernel scaffold; baseline (speedup 1.0000x reference)
import jax
import jax.numpy as jnp
from jax.experimental import pallas as pl


def kernel(x, gate_w, gate_b):
    raise NotImplementedError("write your pallas kernel here")


# speedup vs baseline: 3.9960x; 3.9960x over previous
import functools

import jax
import jax.numpy as jnp
from jax.experimental import pallas as pl
from jax.experimental.pallas import tpu as pltpu

BLOCK_M = 1024
NEG_INF = float("-inf")


def _router_kernel(x_ref, w_ref, b_ref, gates_ref, logits_ref):
    logits = (
        jnp.dot(x_ref[...], w_ref[...], preferred_element_type=jnp.float32)
        + b_ref[...]
    )
    logits_ref[...] = logits

    n = logits.shape[-1]
    col = jax.lax.broadcasted_iota(jnp.int32, logits.shape, 1)

    v1 = jnp.max(logits, axis=-1, keepdims=True)
    i1 = jnp.argmax(logits, axis=-1)[:, None]
    masked = jnp.where(col == i1, NEG_INF, logits)
    v2 = jnp.max(masked, axis=-1, keepdims=True)
    i2 = jnp.argmax(masked, axis=-1)[:, None]

    p1 = 1.0 / (1.0 + jnp.exp(v2 - v1))
    p2 = 1.0 - p1

    gates_ref[...] = jnp.where(col == i1, p1, jnp.where(col == i2, p2, 0.0))


@jax.jit
def kernel(x, gate_w, gate_b):
    m, k = x.shape
    n = gate_w.shape[1]
    grid = (m // BLOCK_M,)
    gates, logits = pl.pallas_call(
        _router_kernel,
        grid=grid,
        in_specs=[
            pl.BlockSpec((BLOCK_M, k), lambda i: (i, 0)),
            pl.BlockSpec((k, n), lambda i: (0, 0)),
            pl.BlockSpec((1, n), lambda i: (0, 0)),
        ],
        out_specs=[
            pl.BlockSpec((BLOCK_M, n), lambda i: (i, 0)),
            pl.BlockSpec((BLOCK_M, n), lambda i: (i, 0)),
        ],
        out_shape=[
            jax.ShapeDtypeStruct((m, n), jnp.float32),
            jax.ShapeDtypeStruct((m, n), jnp.float32),
        ],
    )(x, gate_w, gate_b.reshape(1, n))
    return (gates, logits)


# speedup vs baseline: 4.0283x; 1.0081x over previous
import functools

import jax
import jax.numpy as jnp
from jax.experimental import pallas as pl
from jax.experimental.pallas import tpu as pltpu

BLOCK_M = 2048
NEG_INF = float("-inf")


def _router_kernel(x_ref, w_ref, b_ref, gates_ref, logits_ref):
    logits = (
        jnp.dot(x_ref[...], w_ref[...], preferred_element_type=jnp.float32)
        + b_ref[...]
    )
    logits_ref[...] = logits

    n = logits.shape[-1]
    col = jax.lax.broadcasted_iota(jnp.int32, logits.shape, 1)

    v1 = jnp.max(logits, axis=-1, keepdims=True)
    i1 = jnp.argmax(logits, axis=-1)[:, None]
    masked = jnp.where(col == i1, NEG_INF, logits)
    v2 = jnp.max(masked, axis=-1, keepdims=True)
    i2 = jnp.argmax(masked, axis=-1)[:, None]

    p1 = 1.0 / (1.0 + jnp.exp(v2 - v1))
    p2 = 1.0 - p1

    gates_ref[...] = jnp.where(col == i1, p1, jnp.where(col == i2, p2, 0.0))


@jax.jit
def kernel(x, gate_w, gate_b):
    m, k = x.shape
    n = gate_w.shape[1]
    grid = (m // BLOCK_M,)
    gates, logits = pl.pallas_call(
        _router_kernel,
        grid=grid,
        in_specs=[
            pl.BlockSpec((BLOCK_M, k), lambda i: (i, 0)),
            pl.BlockSpec((k, n), lambda i: (0, 0)),
            pl.BlockSpec((1, n), lambda i: (0, 0)),
        ],
        out_specs=[
            pl.BlockSpec((BLOCK_M, n), lambda i: (i, 0)),
            pl.BlockSpec((BLOCK_M, n), lambda i: (i, 0)),
        ],
        out_shape=[
            jax.ShapeDtypeStruct((m, n), jnp.float32),
            jax.ShapeDtypeStruct((m, n), jnp.float32),
        ],
        compiler_params=pltpu.CompilerParams(
            dimension_semantics=("parallel",),
        ),
    )(x, gate_w, gate_b.reshape(1, n))
    return (gates, logits)
